# baseline (speedup 1.0000x reference)
import functools

import jax
import jax.numpy as jnp
from jax.experimental import pallas as pl
from jax.experimental.pallas import tpu as pltpu

EPS = 1e-5
Z_COEF = 0.001


def _rmsnorm(v, w):
    return v * jax.lax.rsqrt(jnp.mean(v * v, axis=-1, keepdims=True) + EPS) * w


def _qkv_body(x_ref, ln_ref, w_ref, o_ref):
    h = _rmsnorm(x_ref[...], ln_ref[...])
    o_ref[...] = jnp.dot(h, w_ref[...], preferred_element_type=jnp.float32)


def _qkv(x, ln1_w, Wqkv, *, bm, bn):
    S, D = x.shape
    N = Wqkv.shape[1]
    return pl.pallas_call(
        _qkv_body,
        grid=(S // bm, N // bn),
        in_specs=[
            pl.BlockSpec((bm, D), lambda i, j: (i, 0)),
            pl.BlockSpec((1, D), lambda i, j: (0, 0)),
            pl.BlockSpec((D, bn), lambda i, j: (0, j)),
        ],
        out_specs=pl.BlockSpec((bm, bn), lambda i, j: (i, j)),
        out_shape=jax.ShapeDtypeStruct((S, N), jnp.float32),
    )(x, ln1_w.reshape(1, D), Wqkv)


def _attn_body(q_ref, k_ref, v_ref, o_ref, *, scale):
    q = q_ref[0]
    k = k_ref[0]
    s = jax.lax.dot_general(q, k, (((1,), (1,)), ((), ())),
                            preferred_element_type=jnp.float32) * scale
    m = jnp.max(s, axis=-1, keepdims=True)
    p = jnp.exp(s - m)
    p = p / jnp.sum(p, axis=-1, keepdims=True)
    o_ref[0] = jnp.dot(p, v_ref[0], preferred_element_type=jnp.float32)


def _attention(q, k, v, *, bq):
    H, S, DH = q.shape
    return pl.pallas_call(
        functools.partial(_attn_body, scale=1.0 / (DH ** 0.5)),
        grid=(H, S // bq),
        in_specs=[
            pl.BlockSpec((1, bq, DH), lambda h, i: (h, i, 0)),
            pl.BlockSpec((1, S, DH), lambda h, i: (h, 0, 0)),
            pl.BlockSpec((1, S, DH), lambda h, i: (h, 0, 0)),
        ],
        out_specs=pl.BlockSpec((1, bq, DH), lambda h, i: (h, i, 0)),
        out_shape=jax.ShapeDtypeStruct((H, S, DH), jnp.float32),
    )(q, k, v)


def _post_body(ao_ref, x_ref, wo_ref, ln_ref, wr_ref, x2_ref, h2_ref, lg_ref):
    x2 = x_ref[...] + jnp.dot(ao_ref[...], wo_ref[...],
                              preferred_element_type=jnp.float32)
    h2 = _rmsnorm(x2, ln_ref[...])
    x2_ref[...] = x2
    h2_ref[...] = h2
    lg_ref[...] = jnp.dot(h2, wr_ref[...], preferred_element_type=jnp.float32)


def _post(ao, x, Wo, ln2_w, Wr, *, bm):
    S, D = x.shape
    E = Wr.shape[1]
    return pl.pallas_call(
        _post_body,
        grid=(S // bm,),
        in_specs=[
            pl.BlockSpec((bm, D), lambda i: (i, 0)),
            pl.BlockSpec((bm, D), lambda i: (i, 0)),
            pl.BlockSpec((D, D), lambda i: (0, 0)),
            pl.BlockSpec((1, D), lambda i: (0, 0)),
            pl.BlockSpec((D, E), lambda i: (0, 0)),
        ],
        out_specs=[
            pl.BlockSpec((bm, D), lambda i: (i, 0)),
            pl.BlockSpec((bm, D), lambda i: (i, 0)),
            pl.BlockSpec((bm, E), lambda i: (i, 0)),
        ],
        out_shape=[
            jax.ShapeDtypeStruct((S, D), jnp.float32),
            jax.ShapeDtypeStruct((S, D), jnp.float32),
            jax.ShapeDtypeStruct((S, E), jnp.float32),
        ],
    )(ao, x, Wo, ln2_w.reshape(1, D), Wr)


def _router_body(lg_ref, rw_ref, loss_ref):
    logits = lg_ref[...]
    S, E = logits.shape
    m = jnp.max(logits, axis=-1, keepdims=True)
    ex = jnp.exp(logits - m)
    den = jnp.sum(ex, axis=-1, keepdims=True)
    probs = ex / den

    cols = jax.lax.broadcasted_iota(jnp.int32, (S, E), 1)
    i1 = jnp.argmax(probs, axis=-1)[:, None]
    w1 = jnp.max(probs, axis=-1, keepdims=True)
    masked = jnp.where(cols == i1, -jnp.inf, probs)
    i2 = jnp.argmax(masked, axis=-1)[:, None]
    w2 = jnp.max(masked, axis=-1, keepdims=True)
    tot = w1 + w2
    rw = jnp.where(cols == i1, w1 / tot, 0.0) + jnp.where(cols == i2, w2 / tot, 0.0)
    rw_ref[...] = rw

    z = jnp.log(den[:, 0]) + m[:, 0]
    z_loss = Z_COEF * jnp.mean(z * z)
    f = jnp.mean((rw > 0).astype(jnp.float32), axis=0)
    P = jnp.mean(probs, axis=0)
    loss_ref[...] = (E * jnp.sum(f * P) + z_loss).reshape(1, 1)


def _router(logits):
    S, E = logits.shape
    return pl.pallas_call(
        _router_body,
        out_shape=[
            jax.ShapeDtypeStruct((S, E), jnp.float32),
            jax.ShapeDtypeStruct((1, 1), jnp.float32),
        ],
    )(logits)


def _moe_body(h2_ref, rwt_ref, x2_ref, w1_ref, w2_ref, o_ref):
    e = pl.program_id(1)
    f = pl.program_id(2)
    h = jnp.dot(h2_ref[...], w1_ref[0], preferred_element_type=jnp.float32)
    h = jax.nn.gelu(h)
    c = jnp.dot(h, w2_ref[0], preferred_element_type=jnp.float32)
    c = c * rwt_ref[0, 0][:, None]

    @pl.when(jnp.logical_and(e == 0, f == 0))
    def _():
        o_ref[...] = x2_ref[...] + c

    @pl.when(jnp.logical_or(e != 0, f != 0))
    def _():
        o_ref[...] += c


def _moe(h2, rw_t, x2, W1, W2, *, bm, bf):
    S, D = h2.shape
    E, _, F = W1.shape
    return pl.pallas_call(
        _moe_body,
        grid=(S // bm, E, F // bf),
        in_specs=[
            pl.BlockSpec((bm, D), lambda i, e, f: (i, 0)),
            pl.BlockSpec((1, 1, bm), lambda i, e, f: (e, 0, i)),
            pl.BlockSpec((bm, D), lambda i, e, f: (i, 0)),
            pl.BlockSpec((1, D, bf), lambda i, e, f: (e, 0, f)),
            pl.BlockSpec((1, bf, D), lambda i, e, f: (e, f, 0)),
        ],
        out_specs=pl.BlockSpec((bm, D), lambda i, e, f: (i, 0)),
        out_shape=jax.ShapeDtypeStruct((S, D), jnp.float32),
    )(h2, rw_t, x2, W1, W2)


def kernel(x, ln1_w, ln2_w, Wq, Wk, Wv, Wo, Wr, W1, W2):
    B, S, D = x.shape
    E = Wr.shape[1]
    H = 16
    DH = D // H
    xs = x.reshape(S, D)

    Wqkv = jnp.concatenate([Wq, Wk, Wv], axis=1)
    qkv = _qkv(xs, ln1_w, Wqkv, bm=256, bn=512)
    q, k, v = jnp.split(qkv, 3, axis=1)
    q = q.reshape(S, H, DH).transpose(1, 0, 2)
    k = k.reshape(S, H, DH).transpose(1, 0, 2)
    v = v.reshape(S, H, DH).transpose(1, 0, 2)
    ao = _attention(q, k, v, bq=256)
    ao = ao.transpose(1, 0, 2).reshape(S, D)

    x2, h2, logits = _post(ao, xs, Wo, ln2_w, Wr, bm=256)
    rw, loss = _router(logits)
    rw_t = rw.T.reshape(E, 1, S)
    out = _moe(h2, rw_t, x2, W1, W2, bm=256, bf=512)
    return (out.reshape(B, S, D), loss.reshape(()))


# speedup vs baseline: 1.2091x; 1.2091x over previous
import functools

import jax
import jax.numpy as jnp
from jax.experimental import pallas as pl
from jax.experimental.pallas import tpu as pltpu

EPS = 1e-5
Z_COEF = 0.001


def _rmsnorm(v, w):
    return v * jax.lax.rsqrt(jnp.mean(v * v, axis=-1, keepdims=True) + EPS) * w


def _qkv_body(x_ref, ln_ref, w_ref, o_ref):
    h = _rmsnorm(x_ref[...], ln_ref[...]).astype(jnp.bfloat16)
    o_ref[...] = jnp.dot(h, w_ref[...],
                         preferred_element_type=jnp.float32).astype(jnp.bfloat16)


def _qkv(x, ln1_w, Wqkv, *, bm, bn):
    S, D = x.shape
    N = Wqkv.shape[1]
    return pl.pallas_call(
        _qkv_body,
        grid=(S // bm, N // bn),
        in_specs=[
            pl.BlockSpec((bm, D), lambda i, j: (i, 0)),
            pl.BlockSpec((1, D), lambda i, j: (0, 0)),
            pl.BlockSpec((D, bn), lambda i, j: (0, j)),
        ],
        out_specs=pl.BlockSpec((bm, bn), lambda i, j: (i, j)),
        out_shape=jax.ShapeDtypeStruct((S, N), jnp.bfloat16),
    )(x, ln1_w.reshape(1, D), Wqkv)


def _attn_body(q_ref, k_ref, v_ref, o_ref, *, scale):
    q = q_ref[0]
    k = k_ref[0]
    s = jax.lax.dot_general(q, k, (((1,), (1,)), ((), ())),
                            preferred_element_type=jnp.float32) * scale
    m = jnp.max(s, axis=-1, keepdims=True)
    p = jnp.exp(s - m)
    p = (p / jnp.sum(p, axis=-1, keepdims=True)).astype(jnp.bfloat16)
    o_ref[0] = jnp.dot(p, v_ref[0],
                       preferred_element_type=jnp.float32).astype(jnp.bfloat16)


def _attention(q, k, v, *, bq):
    H, S, DH = q.shape
    return pl.pallas_call(
        functools.partial(_attn_body, scale=1.0 / (DH ** 0.5)),
        grid=(H, S // bq),
        in_specs=[
            pl.BlockSpec((1, bq, DH), lambda h, i: (h, i, 0)),
            pl.BlockSpec((1, S, DH), lambda h, i: (h, 0, 0)),
            pl.BlockSpec((1, S, DH), lambda h, i: (h, 0, 0)),
        ],
        out_specs=pl.BlockSpec((1, bq, DH), lambda h, i: (h, i, 0)),
        out_shape=jax.ShapeDtypeStruct((H, S, DH), jnp.bfloat16),
    )(q, k, v)


def _post_body(ao_ref, x_ref, wo_ref, ln_ref, wr_ref, x2_ref, h2_ref, lg_ref):
    x2 = x_ref[...] + jnp.dot(ao_ref[...], wo_ref[...],
                              preferred_element_type=jnp.float32)
    h2 = _rmsnorm(x2, ln_ref[...])
    x2_ref[...] = x2
    h2_ref[...] = h2.astype(jnp.bfloat16)
    lg_ref[...] = jnp.dot(h2, wr_ref[...], preferred_element_type=jnp.float32)


def _post(ao, x, Wo, ln2_w, Wr, *, bm):
    S, D = x.shape
    E = Wr.shape[1]
    return pl.pallas_call(
        _post_body,
        grid=(S // bm,),
        in_specs=[
            pl.BlockSpec((bm, D), lambda i: (i, 0)),
            pl.BlockSpec((bm, D), lambda i: (i, 0)),
            pl.BlockSpec((D, D), lambda i: (0, 0)),
            pl.BlockSpec((1, D), lambda i: (0, 0)),
            pl.BlockSpec((D, E), lambda i: (0, 0)),
        ],
        out_specs=[
            pl.BlockSpec((bm, D), lambda i: (i, 0)),
            pl.BlockSpec((bm, D), lambda i: (i, 0)),
            pl.BlockSpec((bm, E), lambda i: (i, 0)),
        ],
        out_shape=[
            jax.ShapeDtypeStruct((S, D), jnp.float32),
            jax.ShapeDtypeStruct((S, D), jnp.bfloat16),
            jax.ShapeDtypeStruct((S, E), jnp.float32),
        ],
    )(ao, x, Wo, ln2_w.reshape(1, D), Wr)


def _router_body(lg_ref, rw_ref, loss_ref):
    logits = lg_ref[...]
    S, E = logits.shape
    m = jnp.max(logits, axis=-1, keepdims=True)
    ex = jnp.exp(logits - m)
    den = jnp.sum(ex, axis=-1, keepdims=True)
    probs = ex / den

    cols = jax.lax.broadcasted_iota(jnp.int32, (S, E), 1)
    i1 = jnp.argmax(probs, axis=-1)[:, None]
    w1 = jnp.max(probs, axis=-1, keepdims=True)
    masked = jnp.where(cols == i1, -jnp.inf, probs)
    i2 = jnp.argmax(masked, axis=-1)[:, None]
    w2 = jnp.max(masked, axis=-1, keepdims=True)
    tot = w1 + w2
    rw = jnp.where(cols == i1, w1 / tot, 0.0) + jnp.where(cols == i2, w2 / tot, 0.0)
    rw_ref[...] = rw

    z = jnp.log(den[:, 0]) + m[:, 0]
    z_loss = Z_COEF * jnp.mean(z * z)
    f = jnp.mean((rw > 0).astype(jnp.float32), axis=0)
    P = jnp.mean(probs, axis=0)
    loss_ref[...] = (E * jnp.sum(f * P) + z_loss).reshape(1, 1)


def _router(logits):
    S, E = logits.shape
    return pl.pallas_call(
        _router_body,
        out_shape=[
            jax.ShapeDtypeStruct((S, E), jnp.float32),
            jax.ShapeDtypeStruct((1, 1), jnp.float32),
        ],
    )(logits)


def _moe_body(h2_ref, rwt_ref, x2_ref, w1_ref, w2_ref, o_ref):
    e = pl.program_id(1)
    f = pl.program_id(2)
    h = jnp.dot(h2_ref[...], w1_ref[0], preferred_element_type=jnp.float32)
    h = jax.nn.gelu(h).astype(jnp.bfloat16)
    c = jnp.dot(h, w2_ref[0], preferred_element_type=jnp.float32)
    c = c * rwt_ref[0, 0][:, None]

    @pl.when(jnp.logical_and(e == 0, f == 0))
    def _():
        o_ref[...] = x2_ref[...] + c

    @pl.when(jnp.logical_or(e != 0, f != 0))
    def _():
        o_ref[...] += c


def _moe(h2, rw_t, x2, W1, W2, *, bm, bf):
    S, D = h2.shape
    E, _, F = W1.shape
    return pl.pallas_call(
        _moe_body,
        grid=(S // bm, E, F // bf),
        in_specs=[
            pl.BlockSpec((bm, D), lambda i, e, f: (i, 0)),
            pl.BlockSpec((1, 1, bm), lambda i, e, f: (e, 0, i)),
            pl.BlockSpec((bm, D), lambda i, e, f: (i, 0)),
            pl.BlockSpec((1, D, bf), lambda i, e, f: (e, 0, f)),
            pl.BlockSpec((1, bf, D), lambda i, e, f: (e, f, 0)),
        ],
        out_specs=pl.BlockSpec((bm, D), lambda i, e, f: (i, 0)),
        out_shape=jax.ShapeDtypeStruct((S, D), jnp.float32),
    )(h2, rw_t, x2, W1, W2)


def kernel(x, ln1_w, ln2_w, Wq, Wk, Wv, Wo, Wr, W1, W2):
    B, S, D = x.shape
    E = Wr.shape[1]
    H = 16
    DH = D // H
    xs = x.reshape(S, D)

    Wqkv = jnp.concatenate([Wq, Wk, Wv], axis=1).astype(jnp.bfloat16)
    Wo = Wo.astype(jnp.bfloat16)
    W1 = W1.astype(jnp.bfloat16)
    W2 = W2.astype(jnp.bfloat16)
    qkv = _qkv(xs, ln1_w, Wqkv, bm=256, bn=512)
    q, k, v = jnp.split(qkv, 3, axis=1)
    q = q.reshape(S, H, DH).transpose(1, 0, 2)
    k = k.reshape(S, H, DH).transpose(1, 0, 2)
    v = v.reshape(S, H, DH).transpose(1, 0, 2)
    ao = _attention(q, k, v, bq=256)
    ao = ao.transpose(1, 0, 2).reshape(S, D)

    x2, h2, logits = _post(ao, xs, Wo, ln2_w, Wr, bm=256)
    rw, loss = _router(logits)
    rw_t = rw.T.reshape(E, 1, S)
    out = _moe(h2, rw_t, x2, W1, W2, bm=256, bf=512)
    return (out.reshape(B, S, D), loss.reshape(()))


# speedup vs baseline: 1.6239x; 1.3431x over previous
import functools

import jax
import jax.numpy as jnp
from jax import lax
from jax.experimental import pallas as pl
from jax.experimental.pallas import tpu as pltpu
from jax.experimental.pallas import tpu_sc as plsc

EPS = 1e-5
Z_COEF = 0.001

S, D, H, E, F = 2048, 1024, 16, 8, 1536
DH = D // H
BM = 256
NBLK = 24
NBLK_PAD = 32
RPAD = NBLK * BM
NC, NS = 2, 16
NW = NC * NS
TPW = S // NW


def _rmsnorm(v, w):
    return v * lax.rsqrt(jnp.mean(v * v, axis=-1, keepdims=True) + EPS) * w


def _qkv_body(x_ref, ln_ref, w_ref, o_ref):
    h = _rmsnorm(x_ref[...], ln_ref[...]).astype(jnp.bfloat16)
    o_ref[...] = jnp.dot(h, w_ref[...],
                         preferred_element_type=jnp.float32).astype(jnp.bfloat16)


def _qkv(x, ln1_w, Wqkv, *, bm, bn):
    N = Wqkv.shape[1]
    return pl.pallas_call(
        _qkv_body,
        grid=(S // bm, N // bn),
        in_specs=[
            pl.BlockSpec((bm, D), lambda i, j: (i, 0)),
            pl.BlockSpec((1, D), lambda i, j: (0, 0)),
            pl.BlockSpec((D, bn), lambda i, j: (0, j)),
        ],
        out_specs=pl.BlockSpec((bm, bn), lambda i, j: (i, j)),
        out_shape=jax.ShapeDtypeStruct((S, N), jnp.bfloat16),
    )(x, ln1_w.reshape(1, D), Wqkv)


def _attn_body(q_ref, k_ref, v_ref, o_ref, *, scale):
    q = q_ref[0]
    k = k_ref[0]
    s = lax.dot_general(q, k, (((1,), (1,)), ((), ())),
                        preferred_element_type=jnp.float32) * scale
    m = jnp.max(s, axis=-1, keepdims=True)
    p = jnp.exp(s - m)
    den = jnp.sum(p, axis=-1, keepdims=True)
    o = jnp.dot(p.astype(jnp.bfloat16), v_ref[0],
                preferred_element_type=jnp.float32)
    o_ref[0] = (o / den).astype(jnp.bfloat16)


def _attention(q, k, v, *, bq):
    return pl.pallas_call(
        functools.partial(_attn_body, scale=1.0 / (DH ** 0.5)),
        grid=(H, S // bq),
        in_specs=[
            pl.BlockSpec((1, bq, DH), lambda h, i: (h, i, 0)),
            pl.BlockSpec((1, S, DH), lambda h, i: (h, 0, 0)),
            pl.BlockSpec((1, S, DH), lambda h, i: (h, 0, 0)),
        ],
        out_specs=pl.BlockSpec((1, bq, DH), lambda h, i: (h, i, 0)),
        out_shape=jax.ShapeDtypeStruct((H, S, DH), jnp.bfloat16),
    )(q, k, v)


def _post_body(ao_ref, x_ref, wo_ref, ln_ref, wr_ref, x2_ref, h2_ref, lg_ref):
    x2 = x_ref[...] + jnp.dot(ao_ref[...], wo_ref[...],
                              preferred_element_type=jnp.float32)
    h2 = _rmsnorm(x2, ln_ref[...])
    x2_ref[...] = x2
    h2_ref[...] = h2
    lg_ref[...] = jnp.dot(h2, wr_ref[...], preferred_element_type=jnp.float32)


def _post(ao, x, Wo, ln2_w, Wr, *, bm):
    return pl.pallas_call(
        _post_body,
        grid=(S // bm,),
        in_specs=[
            pl.BlockSpec((bm, D), lambda i: (i, 0)),
            pl.BlockSpec((bm, D), lambda i: (i, 0)),
            pl.BlockSpec((D, D), lambda i: (0, 0)),
            pl.BlockSpec((1, D), lambda i: (0, 0)),
            pl.BlockSpec((D, E), lambda i: (0, 0)),
        ],
        out_specs=[
            pl.BlockSpec((bm, D), lambda i: (i, 0)),
            pl.BlockSpec((bm, D), lambda i: (i, 0)),
            pl.BlockSpec((bm, E), lambda i: (i, 0)),
        ],
        out_shape=[
            jax.ShapeDtypeStruct((S, D), jnp.float32),
            jax.ShapeDtypeStruct((S, D), jnp.float32),
            jax.ShapeDtypeStruct((S, E), jnp.float32),
        ],
    )(ao, x, Wo, ln2_w.reshape(1, D), Wr)


def _router_body(lg_ref, w1_ref, w2_ref, d1_ref, d2_ref, be_ref, loss_ref):
    logits = lg_ref[...]
    m = jnp.max(logits, axis=-1, keepdims=True)
    ex = jnp.exp(logits - m)
    den = jnp.sum(ex, axis=-1, keepdims=True)
    probs = ex / den

    cols = lax.broadcasted_iota(jnp.int32, (S, E), 1)
    i1 = jnp.argmax(probs, axis=-1)[:, None]
    w1 = jnp.max(probs, axis=-1, keepdims=True)
    oh1 = cols == i1
    masked = jnp.where(oh1, -jnp.inf, probs)
    i2 = jnp.argmax(masked, axis=-1)[:, None]
    w2 = jnp.max(masked, axis=-1, keepdims=True)
    oh2 = cols == i2
    tot = w1 + w2
    w1_ref[...] = w1 / tot
    w2_ref[...] = w2 / tot

    ind = jnp.logical_or(oh1, oh2).astype(jnp.bfloat16)

    C = 256
    r_io = lax.broadcasted_iota(jnp.int32, (C, C), 0)
    c_io = lax.broadcasted_iota(jnp.int32, (C, C), 1)
    tril = (c_io < r_io).astype(jnp.bfloat16)
    base = jnp.zeros((1, E), jnp.float32)
    chunks = []
    for i in range(S // C):
        ic = lax.slice(ind, (i * C, 0), ((i + 1) * C, E))
        chunks.append(jnp.dot(tril, ic, preferred_element_type=jnp.float32)
                      + base)
        base = base + jnp.sum(ic.astype(jnp.float32), axis=0, keepdims=True)
    pos = jnp.concatenate(chunks, axis=0)
    counts = base

    nbp = jnp.ceil(counts / BM) * BM
    e_r = lax.broadcasted_iota(jnp.int32, (E, E), 0)
    e_c = lax.broadcasted_iota(jnp.int32, (E, E), 1)
    triu = (e_r < e_c).astype(jnp.float32)
    off = jnp.dot(nbp, triu, preferred_element_type=jnp.float32)

    dest = off + pos
    d1_ref[...] = jnp.sum(jnp.where(oh1, dest, 0.0), axis=1,
                          keepdims=True).astype(jnp.int32)
    d2_ref[...] = jnp.sum(jnp.where(oh2, dest, 0.0), axis=1,
                          keepdims=True).astype(jnp.int32)

    bstart = lax.broadcasted_iota(jnp.int32, (NBLK_PAD, 1), 0).astype(
        jnp.float32) * BM
    inblk = jnp.logical_and(bstart >= off, bstart < off + nbp)
    ecols = lax.broadcasted_iota(jnp.int32, (NBLK_PAD, E), 1).astype(jnp.float32)
    be = jnp.sum(jnp.where(inblk, ecols + 1.0, 0.0), axis=1, keepdims=True) - 1.0
    be_ref[...] = be.astype(jnp.int32)

    z = jnp.log(den[:, 0]) + m[:, 0]
    z_loss = Z_COEF * jnp.mean(z * z)
    f = counts[0] / S
    P = jnp.mean(probs, axis=0)
    loss_ref[...] = (E * jnp.sum(f * P) + z_loss).reshape(1, 1)


def _router(logits):
    return pl.pallas_call(
        _router_body,
        out_shape=[
            jax.ShapeDtypeStruct((S, 1), jnp.float32),
            jax.ShapeDtypeStruct((S, 1), jnp.float32),
            jax.ShapeDtypeStruct((S, 1), jnp.int32),
            jax.ShapeDtypeStruct((S, 1), jnp.int32),
            jax.ShapeDtypeStruct((NBLK_PAD, 1), jnp.int32),
            jax.ShapeDtypeStruct((1, 1), jnp.float32),
        ],
    )(logits)


def _sc_mesh():
    return plsc.VectorSubcoreMesh(core_axis_name="c", subcore_axis_name="s")


def _sc_dispatch(h2, d1, d2):
    @functools.partial(
        pl.kernel,
        out_type=jax.ShapeDtypeStruct((RPAD, D), jnp.float32),
        mesh=_sc_mesh(),
        scratch_types=[
            pltpu.VMEM((TPW,), jnp.int32),
            pltpu.VMEM((TPW, D), jnp.float32),
            pltpu.SemaphoreType.DMA,
        ],
    )
    def dispatch(h2_hbm, d1_hbm, d2_hbm, sorted_hbm, idx_v, rows_v, sem):
        wid = lax.axis_index("s") * NC + lax.axis_index("c")
        base = wid * TPW
        pltpu.sync_copy(h2_hbm.at[pl.ds(base, TPW)], rows_v)
        pltpu.sync_copy(d1_hbm.at[pl.ds(base, TPW)], idx_v)
        pltpu.async_copy(rows_v, sorted_hbm.at[idx_v], sem).wait()
        pltpu.sync_copy(d2_hbm.at[pl.ds(base, TPW)], idx_v)
        pltpu.async_copy(rows_v, sorted_hbm.at[idx_v], sem).wait()

    return dispatch(h2, d1, d2)


def _moe_body(be_ref, x_ref, w1_ref, w2_ref, y_ref):
    b = pl.program_id(0)

    @pl.when(be_ref[b] >= 0)
    def _():
        xb = x_ref[...].astype(jnp.bfloat16)
        h = jnp.dot(xb, w1_ref[0], preferred_element_type=jnp.float32)
        h = jax.nn.gelu(h).astype(jnp.bfloat16)
        y_ref[...] = jnp.dot(h, w2_ref[0], preferred_element_type=jnp.float32)


def _moe(be, sorted_x, W1, W2):
    grid_spec = pltpu.PrefetchScalarGridSpec(
        num_scalar_prefetch=1,
        grid=(NBLK,),
        in_specs=[
            pl.BlockSpec((BM, D), lambda b, be: (b, 0)),
            pl.BlockSpec((1, D, F), lambda b, be: (jnp.maximum(be[b], 0), 0, 0)),
            pl.BlockSpec((1, F, D), lambda b, be: (jnp.maximum(be[b], 0), 0, 0)),
        ],
        out_specs=pl.BlockSpec((BM, D), lambda b, be: (b, 0)),
    )
    return pl.pallas_call(
        _moe_body,
        grid_spec=grid_spec,
        out_shape=jax.ShapeDtypeStruct((RPAD, D), jnp.float32),
    )(be, sorted_x, W1, W2)


def _sc_collect(y, d1, d2):
    @functools.partial(
        pl.kernel,
        out_type=(jax.ShapeDtypeStruct((S, D), jnp.float32),
                  jax.ShapeDtypeStruct((S, D), jnp.float32)),
        mesh=_sc_mesh(),
        scratch_types=[
            pltpu.VMEM((TPW,), jnp.int32),
            pltpu.VMEM((TPW, D), jnp.float32),
            pltpu.SemaphoreType.DMA,
        ],
    )
    def collect(y_hbm, d1_hbm, d2_hbm, y1_hbm, y2_hbm, idx_v, rows_v, sem):
        wid = lax.axis_index("s") * NC + lax.axis_index("c")
        base = wid * TPW
        pltpu.sync_copy(d1_hbm.at[pl.ds(base, TPW)], idx_v)
        pltpu.async_copy(y_hbm.at[idx_v], rows_v, sem).wait()
        pltpu.sync_copy(rows_v, y1_hbm.at[pl.ds(base, TPW)])
        pltpu.sync_copy(d2_hbm.at[pl.ds(base, TPW)], idx_v)
        pltpu.async_copy(y_hbm.at[idx_v], rows_v, sem).wait()
        pltpu.sync_copy(rows_v, y2_hbm.at[pl.ds(base, TPW)])

    return collect(y, d1, d2)


def _combine_body(x2_ref, y1_ref, y2_ref, w1_ref, w2_ref, o_ref):
    o_ref[...] = (x2_ref[...] + w1_ref[...] * y1_ref[...]
                  + w2_ref[...] * y2_ref[...])


def _combine(x2, y1, y2, w1n, w2n, *, bm):
    return pl.pallas_call(
        _combine_body,
        grid=(S // bm,),
        in_specs=[
            pl.BlockSpec((bm, D), lambda i: (i, 0)),
            pl.BlockSpec((bm, D), lambda i: (i, 0)),
            pl.BlockSpec((bm, D), lambda i: (i, 0)),
            pl.BlockSpec((bm, 1), lambda i: (i, 0)),
            pl.BlockSpec((bm, 1), lambda i: (i, 0)),
        ],
        out_specs=pl.BlockSpec((bm, D), lambda i: (i, 0)),
        out_shape=jax.ShapeDtypeStruct((S, D), jnp.float32),
    )(x2, y1, y2, w1n, w2n)


def kernel(x, ln1_w, ln2_w, Wq, Wk, Wv, Wo, Wr, W1, W2):
    B = x.shape[0]
    xs = x.reshape(S, D)

    Wqkv = jnp.concatenate([Wq, Wk, Wv], axis=1).astype(jnp.bfloat16)
    Wo = Wo.astype(jnp.bfloat16)
    W1 = W1.astype(jnp.bfloat16)
    W2 = W2.astype(jnp.bfloat16)

    qkv = _qkv(xs, ln1_w, Wqkv, bm=256, bn=512)
    q, k, v = jnp.split(qkv, 3, axis=1)
    q = q.reshape(S, H, DH).transpose(1, 0, 2)
    k = k.reshape(S, H, DH).transpose(1, 0, 2)
    v = v.reshape(S, H, DH).transpose(1, 0, 2)
    ao = _attention(q, k, v, bq=256)
    ao = ao.transpose(1, 0, 2).reshape(S, D)

    x2, h2, logits = _post(ao, xs, Wo, ln2_w, Wr, bm=256)
    w1n, w2n, d1, d2, be, loss = _router(logits)
    d1r = d1.reshape(S)
    d2r = d2.reshape(S)

    sorted_x = _sc_dispatch(h2, d1r, d2r)
    y = _moe(be.reshape(NBLK_PAD), sorted_x, W1, W2)
    y1, y2 = _sc_collect(y, d1r, d2r)

    out = _combine(x2, y1, y2, w1n, w2n, bm=256)
    return (out.reshape(B, S, D), loss.reshape(()))
